# baseline (speedup 1.0000x reference)
import functools

import jax
import jax.numpy as jnp
from jax.experimental import pallas as pl
from jax.experimental.pallas import tpu as pltpu

B = 4096
INPUT_DIM = 768
LATENT_DIM = 32
N_EMBED = 256
N_LEVELS = 3
BETA = 0.25

BLOCK_B = 512
N_BLOCKS = B // BLOCK_B
IDX_PAD = 128


def _rqvae_kernel(x_ref, we1, we2, we3, we4, wd1, wd2, wd3, wd4,
                  cb_hi_ref, cb_lo1_ref, cb_lo2_ref,
                  recon_ref, idx_ref, loss_ref, q_ref, idxs_ref):
    i = pl.program_id(0)

    @pl.when(i == 0)
    def _init():
        loss_ref[0, 0] = jnp.float32(0.0)

    def mm(a, b):
        return jax.lax.dot_general(a, b, (((1,), (0,)), ((), ())),
                                   preferred_element_type=jnp.float32)

    def silu(v):
        return v * jax.nn.sigmoid(v)

    @pl.when(i > 0)
    def _decode():
        h = silu(mm(q_ref[...], wd1[...]))
        h = silu(mm(h, wd2[...]))
        h = silu(mm(h, wd3[...]))
        recon_ref[...] = mm(h, wd4[...])
        idx_ref[...] = idxs_ref[...]

    @pl.when(i < N_BLOCKS)
    def _encode():
        h = silu(mm(x_ref[...], we1[...]))
        h = silu(mm(h, we2[...]))
        h = silu(mm(h, we3[...]))
        z = mm(h, we4[...])

        residual = z
        loss_sum = jnp.float32(0.0)
        lane = jax.lax.broadcasted_iota(jnp.int32, (BLOCK_B, IDX_PAD), 1)
        idx_cols = jnp.zeros((BLOCK_B, IDX_PAD), dtype=jnp.int32)
        for lvl in range(N_LEVELS):
            e_hi, e_lo1, e_lo2 = cb_hi_ref[lvl], cb_lo1_ref[lvl], cb_lo2_ref[lvl]
            e = (e_hi + e_lo1) + e_lo2
            d = (jnp.sum(residual * residual, axis=1, keepdims=True)
                 + jnp.sum(e * e, axis=1)[None, :]
                 - 2.0 * mm(residual, e.T))
            idx = jnp.argmin(d, axis=1).astype(jnp.int32)
            onehot = (jax.lax.broadcasted_iota(jnp.int32, (BLOCK_B, N_EMBED), 1)
                      == idx[:, None]).astype(jnp.float32)
            z_q = ((mm(onehot, e_hi) + mm(onehot, e_lo1)) + mm(onehot, e_lo2))
            diff = z_q - residual
            loss_sum = loss_sum + jnp.sum(diff * diff)
            z_q_ste = residual + diff
            residual = residual - z_q_ste
            idx_cols = jnp.where(lane == lvl, idx[:, None], idx_cols)

        q_ref[...] = z - residual
        idxs_ref[...] = idx_cols
        loss_ref[0, 0] += loss_sum / jnp.float32(B * LATENT_DIM)


@jax.jit
def kernel(x, We1, We2, We3, We4, Wd1, Wd2, Wd3, Wd4, codebooks):
    cb_hi = codebooks.astype(jnp.bfloat16).astype(jnp.float32)
    cb_lo1 = (codebooks - cb_hi).astype(jnp.bfloat16).astype(jnp.float32)
    cb_lo2 = (codebooks - cb_hi) - cb_lo1
    full = lambda shape: pl.BlockSpec(shape, lambda i: (0,) * len(shape))
    last = N_BLOCKS - 1
    recon, idx_pad, loss = pl.pallas_call(
        _rqvae_kernel,
        grid=(N_BLOCKS + 1,),
        in_specs=[
            pl.BlockSpec((BLOCK_B, INPUT_DIM),
                         lambda i: (jnp.minimum(i, last), 0)),
            full(We1.shape), full(We2.shape), full(We3.shape), full(We4.shape),
            full(Wd1.shape), full(Wd2.shape), full(Wd3.shape), full(Wd4.shape),
            full(codebooks.shape), full(codebooks.shape), full(codebooks.shape),
        ],
        out_specs=[
            pl.BlockSpec((BLOCK_B, INPUT_DIM),
                         lambda i: (jnp.maximum(i - 1, 0), 0)),
            pl.BlockSpec((BLOCK_B, IDX_PAD),
                         lambda i: (jnp.maximum(i - 1, 0), 0)),
            pl.BlockSpec((1, 1), lambda i: (0, 0), memory_space=pltpu.SMEM),
        ],
        out_shape=[
            jax.ShapeDtypeStruct((B, INPUT_DIM), jnp.float32),
            jax.ShapeDtypeStruct((B, IDX_PAD), jnp.int32),
            jax.ShapeDtypeStruct((1, 1), jnp.float32),
        ],
        scratch_shapes=[
            pltpu.VMEM((BLOCK_B, LATENT_DIM), jnp.float32),
            pltpu.VMEM((BLOCK_B, IDX_PAD), jnp.int32),
        ],
    )(x, We1, We2, We3, We4, Wd1, Wd2, Wd3, Wd4, cb_hi, cb_lo1, cb_lo2)
    loss = loss[0, 0]
    indices = idx_pad[:, :N_LEVELS]
    return recon, loss, loss, (1.0 + BETA) * loss, indices


# speedup vs baseline: 1.0101x; 1.0101x over previous
import functools

import jax
import jax.numpy as jnp
from jax.experimental import pallas as pl
from jax.experimental.pallas import tpu as pltpu

B = 4096
INPUT_DIM = 768
LATENT_DIM = 32
N_EMBED = 256
N_LEVELS = 3
BETA = 0.25

BLOCK_B = 512
N_BLOCKS = B // BLOCK_B
IDX_PAD = 128


def _rqvae_kernel(x_ref, we1, we2, we3, we4, wd1, wd2, wd3, wd4,
                  cb_hi_ref, cb_lo1_ref, cb_lo2_ref,
                  recon_ref, idx_ref, loss_ref):
    i = pl.program_id(0)

    @pl.when(i == 0)
    def _init():
        loss_ref[0, 0] = jnp.float32(0.0)

    def mm(a, b):
        return jax.lax.dot_general(a, b, (((1,), (0,)), ((), ())),
                                   preferred_element_type=jnp.float32)

    def silu(v):
        return v * jax.nn.sigmoid(v)

    h = silu(mm(x_ref[...], we1[...]))
    h = silu(mm(h, we2[...]))
    h = silu(mm(h, we3[...]))
    z = mm(h, we4[...])

    residual = z
    quantized = jnp.zeros_like(z)
    loss_sum = jnp.float32(0.0)
    lane = jax.lax.broadcasted_iota(jnp.int32, (BLOCK_B, IDX_PAD), 1)
    idx_cols = jnp.zeros((BLOCK_B, IDX_PAD), dtype=jnp.int32)
    for lvl in range(N_LEVELS):
        e_hi, e_lo1, e_lo2 = cb_hi_ref[lvl], cb_lo1_ref[lvl], cb_lo2_ref[lvl]
        e = (e_hi + e_lo1) + e_lo2
        d = (jnp.sum(residual * residual, axis=1, keepdims=True)
             + jnp.sum(e * e, axis=1)[None, :]
             - 2.0 * mm(residual, e.T))
        idx = jnp.argmin(d, axis=1).astype(jnp.int32)
        onehot = (jax.lax.broadcasted_iota(jnp.int32, (BLOCK_B, N_EMBED), 1)
                  == idx[:, None]).astype(jnp.float32)
        z_q = ((mm(onehot, e_hi) + mm(onehot, e_lo1)) + mm(onehot, e_lo2))
        diff = z_q - residual
        loss_sum = loss_sum + jnp.sum(diff * diff)
        z_q_ste = residual + diff
        quantized = quantized + z_q_ste
        residual = residual - z_q_ste
        idx_cols = jnp.where(lane == lvl, idx[:, None], idx_cols)

    idx_ref[...] = idx_cols
    loss_ref[0, 0] += loss_sum / jnp.float32(B * LATENT_DIM)

    h = silu(mm(quantized, wd1[...]))
    h = silu(mm(h, wd2[...]))
    h = silu(mm(h, wd3[...]))
    recon_ref[...] = mm(h, wd4[...])


@jax.jit
def kernel(x, We1, We2, We3, We4, Wd1, Wd2, Wd3, Wd4, codebooks):
    cb_hi = codebooks.astype(jnp.bfloat16).astype(jnp.float32)
    cb_lo1 = (codebooks - cb_hi).astype(jnp.bfloat16).astype(jnp.float32)
    cb_lo2 = (codebooks - cb_hi) - cb_lo1
    full = lambda shape: pl.BlockSpec(shape, lambda i: (0,) * len(shape))
    recon, idx_pad, loss = pl.pallas_call(
        _rqvae_kernel,
        grid=(N_BLOCKS,),
        in_specs=[
            pl.BlockSpec((BLOCK_B, INPUT_DIM), lambda i: (i, 0)),
            full(We1.shape), full(We2.shape), full(We3.shape), full(We4.shape),
            full(Wd1.shape), full(Wd2.shape), full(Wd3.shape), full(Wd4.shape),
            full(codebooks.shape), full(codebooks.shape), full(codebooks.shape),
        ],
        out_specs=[
            pl.BlockSpec((BLOCK_B, INPUT_DIM), lambda i: (i, 0)),
            pl.BlockSpec((BLOCK_B, IDX_PAD), lambda i: (i, 0)),
            pl.BlockSpec((1, 1), lambda i: (0, 0), memory_space=pltpu.SMEM),
        ],
        out_shape=[
            jax.ShapeDtypeStruct((B, INPUT_DIM), jnp.float32),
            jax.ShapeDtypeStruct((B, IDX_PAD), jnp.int32),
            jax.ShapeDtypeStruct((1, 1), jnp.float32),
        ],
    )(x, We1, We2, We3, We4, Wd1, Wd2, Wd3, Wd4, cb_hi, cb_lo1, cb_lo2)
    loss = loss[0, 0]
    indices = idx_pad[:, :N_LEVELS]
    return recon, loss, loss, (1.0 + BETA) * loss, indices
